# baseline (speedup 1.0000x reference)
import functools
import jax
import jax.numpy as jnp
import numpy as np
from jax.experimental import pallas as pl
from jax.experimental.pallas import tpu as pltpu


def _index_points(points, idx):
    bidx = jnp.arange(points.shape[0]).reshape((points.shape[0],) + (1,) * (idx.ndim - 1))
    return points[bidx, idx]


def _square_distance(src, dst):
    return (jnp.sum(src ** 2, -1)[:, :, None] + jnp.sum(dst ** 2, -1)[:, None, :]
            - 2.0 * jnp.einsum("bnc,bmc->bnm", src, dst))


def _fps(xyz, npoint):
    b, n, _ = xyz.shape

    def body(i, st):
        cent, dist, far = st
        cent = cent.at[:, i].set(far)
        c = _index_points(xyz, far[:, None])
        d = jnp.sum((xyz - c) ** 2, -1)
        dist = jnp.minimum(dist, d)
        far = jnp.argmax(dist, -1).astype(jnp.int32)
        return cent, dist, far

    cent0 = jnp.zeros((b, npoint), jnp.int32)
    dist0 = jnp.full((b, n), 1e10, jnp.float32)
    far0 = jnp.zeros((b,), jnp.int32)
    cent, _, _ = jax.lax.fori_loop(0, npoint, body, (cent0, dist0, far0))
    return cent


def _query_ball(radius, nsample, xyz, new_xyz):
    b, n, _ = xyz.shape
    s = new_xyz.shape[1]
    sq = _square_distance(new_xyz, xyz)
    gi = jnp.broadcast_to(jnp.arange(n, dtype=jnp.int32), (b, s, n))
    gi = jnp.where(sq > radius * radius, n, gi)
    gi = jnp.sort(gi, -1)[:, :, :nsample]
    first = jnp.broadcast_to(gi[:, :, :1], gi.shape)
    gi = jnp.where(gi == n, first, gi)
    return gi


def _bn(x, g, b_, axes):
    m = jnp.mean(x, axes, keepdims=True)
    v = jnp.var(x, axes, keepdims=True)
    return (x - m) / jnp.sqrt(v + 1e-5) * g + b_


def _sa_msg(xyz_t, pts_t, npoint, radii, nsamples, branches):
    xyz = jnp.transpose(xyz_t, (0, 2, 1))
    pts = jnp.transpose(pts_t, (0, 2, 1))
    new_xyz = _index_points(xyz, _fps(xyz, npoint))
    outs = []
    for r, ns, layers in zip(radii, nsamples, branches):
        idx = _query_ball(r, ns, xyz, new_xyz)
        gx = _index_points(xyz, idx) - new_xyz[:, :, None, :]
        h = jnp.concatenate([_index_points(pts, idx), gx], -1)
        for L in layers:
            h = jax.nn.relu(_bn(h @ L["w"] + L["b"], L["g"], L["bb"], (0, 1, 2)))
        outs.append(jnp.max(h, 2))
    npts = jnp.concatenate(outs, -1)
    return jnp.transpose(new_xyz, (0, 2, 1)), jnp.transpose(npts, (0, 2, 1))


def _fp(xyz1_t, xyz2_t, pts1_t, pts2_t, layers):
    xyz1 = jnp.transpose(xyz1_t, (0, 2, 1))
    xyz2 = jnp.transpose(xyz2_t, (0, 2, 1))
    pts2 = jnp.transpose(pts2_t, (0, 2, 1))
    d = _square_distance(xyz1, xyz2)
    negd, idx = jax.lax.top_k(-d, 3)
    dd = -negd
    rec = 1.0 / (dd + 1e-8)
    w = rec / jnp.sum(rec, -1, keepdims=True)
    interp = jnp.sum(_index_points(pts2, idx) * w[..., None], 2)
    h = jnp.concatenate([jnp.transpose(pts1_t, (0, 2, 1)), interp], -1)
    for L in layers:
        h = jax.nn.relu(_bn(h @ L["w"] + L["b"], L["g"], L["bb"], (0, 1)))
    return jnp.transpose(h, (0, 2, 1))


def _copy_kernel(x_ref, o_ref):
    o_ref[...] = x_ref[...]


def _pallas_copy(x):
    return pl.pallas_call(
        _copy_kernel,
        out_shape=jax.ShapeDtypeStruct(x.shape, x.dtype),
    )(x)


@jax.jit
def kernel(xyz_in, params):
    p = params
    l0_points = xyz_in
    l0_xyz = xyz_in[:, :3, :]
    l1_xyz, l1_points = _sa_msg(l0_xyz, l0_points, 1024, [0.025, 0.05], [32, 64], p["sa1"])
    l2_xyz, l2_points = _sa_msg(l1_xyz, l1_points, 512, [0.05, 0.1], [32, 64], p["sa2"])
    l3_xyz, l3_points = _sa_msg(l2_xyz, l2_points, 256, [0.1, 0.2], [32, 64], p["sa3"])
    l2_points = _fp(l2_xyz, l3_xyz, l2_points, l3_points, p["fp3"])
    l1_points = _fp(l1_xyz, l2_xyz, l1_points, l2_points, p["fp2"])
    l0p = _fp(l0_xyz, l1_xyz, l0_points, l1_points, p["fp1"])
    l0p = _pallas_copy(l0p)
    h = jnp.transpose(l0p, (0, 2, 1))
    o1 = jax.nn.relu(_bn(h @ p["off1"]["w"] + p["off1"]["b"], p["off1"]["g"], p["off1"]["bb"], (0, 1)))
    off = jnp.transpose(o1 @ p["off2"]["w"] + p["off2"]["b"], (0, 2, 1))
    d1 = jax.nn.relu(_bn(h @ p["dist1"]["w"] + p["dist1"]["b"], p["dist1"]["g"], p["dist1"]["bb"], (0, 1)))
    dist = jnp.transpose(d1 @ p["dist2"]["w"] + p["dist2"]["b"], (0, 2, 1))
    c1 = jax.nn.relu(_bn(h @ p["cls1"]["w"] + p["cls1"]["b"], p["cls1"]["g"], p["cls1"]["bb"], (0, 1)))
    cls_pred = jnp.transpose(c1 @ p["cls2"]["w"] + p["cls2"]["b"], (0, 2, 1))
    return (l0p, l3_points, l0_xyz, l3_xyz, off, dist, cls_pred)


# speedup vs baseline: 1.3435x; 1.3435x over previous
import functools
import jax
import jax.numpy as jnp
import numpy as np
from jax.experimental import pallas as pl
from jax.experimental.pallas import tpu as pltpu


def _index_points(points, idx):
    bidx = jnp.arange(points.shape[0]).reshape((points.shape[0],) + (1,) * (idx.ndim - 1))
    return points[bidx, idx]


def _square_distance(src, dst):
    return (jnp.sum(src ** 2, -1)[:, :, None] + jnp.sum(dst ** 2, -1)[:, None, :]
            - 2.0 * jnp.einsum("bnc,bmc->bnm", src, dst))


def _fps_body(xyz_ref, nxyz_ref, *, n, npoint):
    L = n // 8
    M = npoint // 8
    x = xyz_ref[0, 0]
    y = xyz_ref[0, 1]
    z = xyz_ref[0, 2]
    fidx = (jax.lax.broadcasted_iota(jnp.int32, (8, L), 0) * L
            + jax.lax.broadcasted_iota(jnp.int32, (8, L), 1))
    oidx = (jax.lax.broadcasted_iota(jnp.int32, (8, M), 0) * M
            + jax.lax.broadcasted_iota(jnp.int32, (8, M), 1))

    def body(i, st):
        dist, far, cxa, cya, cza = st
        eq = fidx == far
        cx = jnp.sum(jnp.where(eq, x, 0.0))
        cy = jnp.sum(jnp.where(eq, y, 0.0))
        cz = jnp.sum(jnp.where(eq, z, 0.0))
        sel = oidx == i
        cxa = jnp.where(sel, cx, cxa)
        cya = jnp.where(sel, cy, cya)
        cza = jnp.where(sel, cz, cza)
        dx = x - cx
        dy = y - cy
        dz = z - cz
        d = (dx * dx + dy * dy) + dz * dz
        dist = jnp.minimum(dist, d)
        maxv = jnp.max(dist)
        far = jnp.min(jnp.where(dist == maxv, fidx, n))
        return dist, far, cxa, cya, cza

    dist0 = jnp.full((8, L), 1e10, jnp.float32)
    z8 = jnp.zeros((8, M), jnp.float32)
    _, _, cxa, cya, cza = jax.lax.fori_loop(
        0, npoint, body, (dist0, jnp.int32(0), z8, z8, z8))
    nxyz_ref[0, 0] = cxa
    nxyz_ref[0, 1] = cya
    nxyz_ref[0, 2] = cza


def _fps_xyz(xyz_t, npoint):
    b, _, n = xyz_t.shape
    xyz4 = xyz_t.reshape(b, 3, 8, n // 8)
    out = pl.pallas_call(
        functools.partial(_fps_body, n=n, npoint=npoint),
        grid=(b,),
        in_specs=[pl.BlockSpec((1, 3, 8, n // 8), lambda i: (i, 0, 0, 0))],
        out_specs=pl.BlockSpec((1, 3, 8, npoint // 8), lambda i: (i, 0, 0, 0)),
        out_shape=jax.ShapeDtypeStruct((b, 3, 8, npoint // 8), jnp.float32),
    )(xyz4)
    return out.reshape(b, 3, npoint)


def _query_ball(radius, nsample, xyz, new_xyz):
    b, n, _ = xyz.shape
    s = new_xyz.shape[1]
    sq = _square_distance(new_xyz, xyz)
    gi = jnp.broadcast_to(jnp.arange(n, dtype=jnp.int32), (b, s, n))
    gi = jnp.where(sq > radius * radius, n, gi)
    gi = jnp.sort(gi, -1)[:, :, :nsample]
    first = jnp.broadcast_to(gi[:, :, :1], gi.shape)
    gi = jnp.where(gi == n, first, gi)
    return gi


def _bn(x, g, b_, axes):
    m = jnp.mean(x, axes, keepdims=True)
    v = jnp.var(x, axes, keepdims=True)
    return (x - m) / jnp.sqrt(v + 1e-5) * g + b_


def _fps_jnp(xyz, npoint):
    b, n, _ = xyz.shape

    def body(i, st):
        cent, dist, far = st
        cent = cent.at[:, i].set(far)
        c = _index_points(xyz, far[:, None])
        d = jnp.sum((xyz - c) ** 2, -1)
        dist = jnp.minimum(dist, d)
        far = jnp.argmax(dist, -1).astype(jnp.int32)
        return cent, dist, far

    cent0 = jnp.zeros((b, npoint), jnp.int32)
    dist0 = jnp.full((b, n), 1e10, jnp.float32)
    far0 = jnp.zeros((b,), jnp.int32)
    cent, _, _ = jax.lax.fori_loop(0, npoint, body, (cent0, dist0, far0))
    return cent


def _sa_msg(xyz_t, pts_t, npoint, radii, nsamples, branches, use_ref_fps=False):
    xyz = jnp.transpose(xyz_t, (0, 2, 1))
    pts = jnp.transpose(pts_t, (0, 2, 1))
    if use_ref_fps:
        new_xyz = _index_points(xyz, _fps_jnp(xyz, npoint))
    else:
        new_xyz = jnp.transpose(_fps_xyz(xyz_t, npoint), (0, 2, 1))
    outs = []
    for r, ns, layers in zip(radii, nsamples, branches):
        idx = _query_ball(r, ns, xyz, new_xyz)
        gx = _index_points(xyz, idx) - new_xyz[:, :, None, :]
        h = jnp.concatenate([_index_points(pts, idx), gx], -1)
        for L in layers:
            h = jax.nn.relu(_bn(h @ L["w"] + L["b"], L["g"], L["bb"], (0, 1, 2)))
        outs.append(jnp.max(h, 2))
    npts = jnp.concatenate(outs, -1)
    return jnp.transpose(new_xyz, (0, 2, 1)), jnp.transpose(npts, (0, 2, 1))


def _fp(xyz1_t, xyz2_t, pts1_t, pts2_t, layers):
    xyz1 = jnp.transpose(xyz1_t, (0, 2, 1))
    xyz2 = jnp.transpose(xyz2_t, (0, 2, 1))
    pts2 = jnp.transpose(pts2_t, (0, 2, 1))
    d = _square_distance(xyz1, xyz2)
    negd, idx = jax.lax.top_k(-d, 3)
    dd = -negd
    rec = 1.0 / (dd + 1e-8)
    w = rec / jnp.sum(rec, -1, keepdims=True)
    interp = jnp.sum(_index_points(pts2, idx) * w[..., None], 2)
    h = jnp.concatenate([jnp.transpose(pts1_t, (0, 2, 1)), interp], -1)
    for L in layers:
        h = jax.nn.relu(_bn(h @ L["w"] + L["b"], L["g"], L["bb"], (0, 1)))
    return jnp.transpose(h, (0, 2, 1))


def _copy_kernel(x_ref, o_ref):
    o_ref[...] = x_ref[...]


def _pallas_copy(x):
    return pl.pallas_call(
        _copy_kernel,
        out_shape=jax.ShapeDtypeStruct(x.shape, x.dtype),
    )(x)


@jax.jit
def kernel(xyz_in, params):
    p = params
    l0_points = xyz_in
    l0_xyz = xyz_in[:, :3, :]
    l1_xyz, l1_points = _sa_msg(l0_xyz, l0_points, 1024, [0.025, 0.05], [32, 64], p["sa1"])
    l2_xyz, l2_points = _sa_msg(l1_xyz, l1_points, 512, [0.05, 0.1], [32, 64], p["sa2"])
    l3_xyz, l3_points = _sa_msg(l2_xyz, l2_points, 256, [0.1, 0.2], [32, 64], p["sa3"],
                                use_ref_fps=True)
    l2_points = _fp(l2_xyz, l3_xyz, l2_points, l3_points, p["fp3"])
    l1_points = _fp(l1_xyz, l2_xyz, l1_points, l2_points, p["fp2"])
    l0p = _fp(l0_xyz, l1_xyz, l0_points, l1_points, p["fp1"])
    l0p = _pallas_copy(l0p)
    h = jnp.transpose(l0p, (0, 2, 1))
    o1 = jax.nn.relu(_bn(h @ p["off1"]["w"] + p["off1"]["b"], p["off1"]["g"], p["off1"]["bb"], (0, 1)))
    off = jnp.transpose(o1 @ p["off2"]["w"] + p["off2"]["b"], (0, 2, 1))
    d1 = jax.nn.relu(_bn(h @ p["dist1"]["w"] + p["dist1"]["b"], p["dist1"]["g"], p["dist1"]["bb"], (0, 1)))
    dist = jnp.transpose(d1 @ p["dist2"]["w"] + p["dist2"]["b"], (0, 2, 1))
    c1 = jax.nn.relu(_bn(h @ p["cls1"]["w"] + p["cls1"]["b"], p["cls1"]["g"], p["cls1"]["bb"], (0, 1)))
    cls_pred = jnp.transpose(c1 @ p["cls2"]["w"] + p["cls2"]["b"], (0, 2, 1))
    return (l0p, l3_points, l0_xyz, l3_xyz, off, dist, cls_pred)


# speedup vs baseline: 1.3446x; 1.0007x over previous
import functools
import jax
import jax.numpy as jnp
from jax.experimental import pallas as pl


def _index_points(points, idx):
    bidx = jnp.arange(points.shape[0]).reshape((points.shape[0],) + (1,) * (idx.ndim - 1))
    return points[bidx, idx]


def _square_distance(src, dst):
    return (jnp.sum(src ** 2, -1)[:, :, None] + jnp.sum(dst ** 2, -1)[:, None, :]
            - 2.0 * jnp.einsum("bnc,bmc->bnm", src, dst))


def _fps_body(xyz_ref, nxyz_ref, *, n, npoint):
    L = n // 8
    M = npoint // 8
    x = xyz_ref[0, 0]
    y = xyz_ref[0, 1]
    z = xyz_ref[0, 2]
    fidx = (jax.lax.broadcasted_iota(jnp.int32, (8, L), 0) * L
            + jax.lax.broadcasted_iota(jnp.int32, (8, L), 1))
    oidx = (jax.lax.broadcasted_iota(jnp.int32, (8, M), 0) * M
            + jax.lax.broadcasted_iota(jnp.int32, (8, M), 1))

    def body(i, st):
        dist, far, cxa, cya, cza = st
        eq = fidx == far
        cx = jnp.sum(jnp.where(eq, x, 0.0))
        cy = jnp.sum(jnp.where(eq, y, 0.0))
        cz = jnp.sum(jnp.where(eq, z, 0.0))
        sel = oidx == i
        cxa = jnp.where(sel, cx, cxa)
        cya = jnp.where(sel, cy, cya)
        cza = jnp.where(sel, cz, cza)
        dx = x - cx
        dy = y - cy
        dz = z - cz
        d = (dx * dx + dy * dy) + dz * dz
        dist = jnp.minimum(dist, d)
        maxv = jnp.max(dist)
        far = jnp.min(jnp.where(dist == maxv, fidx, n))
        return dist, far, cxa, cya, cza

    dist0 = jnp.full((8, L), 1e10, jnp.float32)
    z8 = jnp.zeros((8, M), jnp.float32)
    _, _, cxa, cya, cza = jax.lax.fori_loop(
        0, npoint, body, (dist0, jnp.int32(0), z8, z8, z8))
    nxyz_ref[0, 0] = cxa
    nxyz_ref[0, 1] = cya
    nxyz_ref[0, 2] = cza


def _fps_xyz(xyz_t, npoint):
    b, _, n = xyz_t.shape
    xyz4 = xyz_t.reshape(b, 3, 8, n // 8)
    out = pl.pallas_call(
        functools.partial(_fps_body, n=n, npoint=npoint),
        grid=(b,),
        in_specs=[pl.BlockSpec((1, 3, 8, n // 8), lambda i: (i, 0, 0, 0))],
        out_specs=pl.BlockSpec((1, 3, 8, npoint // 8), lambda i: (i, 0, 0, 0)),
        out_shape=jax.ShapeDtypeStruct((b, 3, 8, npoint // 8), jnp.float32),
    )(xyz4)
    return out.reshape(b, 3, npoint)


def _query_ball(radius, nsample, xyz, new_xyz):
    b, n, _ = xyz.shape
    s = new_xyz.shape[1]
    sq = _square_distance(new_xyz, xyz)
    gi = jnp.broadcast_to(jnp.arange(n, dtype=jnp.int32), (b, s, n))
    gi = jnp.where(sq > radius * radius, n, gi)
    gi = jnp.sort(gi, -1)[:, :, :nsample]
    first = jnp.broadcast_to(gi[:, :, :1], gi.shape)
    gi = jnp.where(gi == n, first, gi)
    return gi


def _bn(x, g, b_, axes):
    m = jnp.mean(x, axes, keepdims=True)
    v = jnp.var(x, axes, keepdims=True)
    return (x - m) / jnp.sqrt(v + 1e-5) * g + b_


def _fps_jnp(xyz, npoint):
    b, n, _ = xyz.shape

    def body(i, st):
        cent, dist, far = st
        cent = cent.at[:, i].set(far)
        c = _index_points(xyz, far[:, None])
        d = jnp.sum((xyz - c) ** 2, -1)
        dist = jnp.minimum(dist, d)
        far = jnp.argmax(dist, -1).astype(jnp.int32)
        return cent, dist, far

    cent0 = jnp.zeros((b, npoint), jnp.int32)
    dist0 = jnp.full((b, n), 1e10, jnp.float32)
    far0 = jnp.zeros((b,), jnp.int32)
    cent, _, _ = jax.lax.fori_loop(0, npoint, body, (cent0, dist0, far0))
    return cent


def _sa_msg(xyz_t, pts_t, npoint, radii, nsamples, branches, use_ref_fps=False):
    xyz = jnp.transpose(xyz_t, (0, 2, 1))
    pts = jnp.transpose(pts_t, (0, 2, 1))
    if use_ref_fps:
        new_xyz = _index_points(xyz, _fps_jnp(xyz, npoint))
    else:
        new_xyz = jnp.transpose(_fps_xyz(xyz_t, npoint), (0, 2, 1))
    outs = []
    for r, ns, layers in zip(radii, nsamples, branches):
        idx = _query_ball(r, ns, xyz, new_xyz)
        gx = _index_points(xyz, idx) - new_xyz[:, :, None, :]
        h = jnp.concatenate([_index_points(pts, idx), gx], -1)
        for L in layers:
            h = jax.nn.relu(_bn(h @ L["w"] + L["b"], L["g"], L["bb"], (0, 1, 2)))
        outs.append(jnp.max(h, 2))
    npts = jnp.concatenate(outs, -1)
    return jnp.transpose(new_xyz, (0, 2, 1)), jnp.transpose(npts, (0, 2, 1))


def _fp(xyz1_t, xyz2_t, pts1_t, pts2_t, layers):
    xyz1 = jnp.transpose(xyz1_t, (0, 2, 1))
    xyz2 = jnp.transpose(xyz2_t, (0, 2, 1))
    pts2 = jnp.transpose(pts2_t, (0, 2, 1))
    d = _square_distance(xyz1, xyz2)
    negd, idx = jax.lax.top_k(-d, 3)
    dd = -negd
    rec = 1.0 / (dd + 1e-8)
    w = rec / jnp.sum(rec, -1, keepdims=True)
    interp = jnp.sum(_index_points(pts2, idx) * w[..., None], 2)
    h = jnp.concatenate([jnp.transpose(pts1_t, (0, 2, 1)), interp], -1)
    for L in layers:
        h = jax.nn.relu(_bn(h @ L["w"] + L["b"], L["g"], L["bb"], (0, 1)))
    return jnp.transpose(h, (0, 2, 1))


def _copy_kernel(x_ref, o_ref):
    o_ref[...] = x_ref[...]


def _pallas_copy(x):
    return pl.pallas_call(
        _copy_kernel,
        out_shape=jax.ShapeDtypeStruct(x.shape, x.dtype),
    )(x)


@jax.jit
def kernel(xyz_in, params):
    p = params
    l0_points = xyz_in
    l0_xyz = xyz_in[:, :3, :]
    l1_xyz, l1_points = _sa_msg(l0_xyz, l0_points, 1024, [0.025, 0.05], [32, 64], p["sa1"])
    l2_xyz, l2_points = _sa_msg(l1_xyz, l1_points, 512, [0.05, 0.1], [32, 64], p["sa2"])
    l3_xyz, l3_points = _sa_msg(l2_xyz, l2_points, 256, [0.1, 0.2], [32, 64], p["sa3"],
                                use_ref_fps=True)
    l2_points = _fp(l2_xyz, l3_xyz, l2_points, l3_points, p["fp3"])
    l1_points = _fp(l1_xyz, l2_xyz, l1_points, l2_points, p["fp2"])
    l0p = _fp(l0_xyz, l1_xyz, l0_points, l1_points, p["fp1"])
    l0p = _pallas_copy(l0p)
    h = jnp.transpose(l0p, (0, 2, 1))
    o1 = jax.nn.relu(_bn(h @ p["off1"]["w"] + p["off1"]["b"], p["off1"]["g"], p["off1"]["bb"], (0, 1)))
    off = jnp.transpose(o1 @ p["off2"]["w"] + p["off2"]["b"], (0, 2, 1))
    d1 = jax.nn.relu(_bn(h @ p["dist1"]["w"] + p["dist1"]["b"], p["dist1"]["g"], p["dist1"]["bb"], (0, 1)))
    dist = jnp.transpose(d1 @ p["dist2"]["w"] + p["dist2"]["b"], (0, 2, 1))
    c1 = jax.nn.relu(_bn(h @ p["cls1"]["w"] + p["cls1"]["b"], p["cls1"]["g"], p["cls1"]["bb"], (0, 1)))
    cls_pred = jnp.transpose(c1 @ p["cls2"]["w"] + p["cls2"]["b"], (0, 2, 1))
    return (l0p, l3_points, l0_xyz, l3_xyz, off, dist, cls_pred)
